# baseline (speedup 1.0000x reference)
import jax
import jax.numpy as jnp
from jax.experimental import pallas as pl
from jax.experimental.pallas import tpu as pltpu

_NLAYER = 3
_L = 24
_LPAD = 32
_NTILE = 8
_XCH = 8
_OCH = 16


def _gnn_body(x_hbm, adj_hbm, ws_ref, wn_ref, b_ref, out_hbm,
              x_vmem, adj_vmem, out_vmem, in_sems, adj_sem, out_sems):
    nG, nN, L = x_vmem.shape
    nS = 4
    nC = 8

    for i in range(_XCH):
        sl = pl.ds(i * (nG // _XCH), nG // _XCH)
        pltpu.make_async_copy(x_hbm.at[sl], x_vmem.at[sl], in_sems.at[i]).start()
    pltpu.make_async_copy(adj_hbm, adj_vmem, adj_sem).start()
    for i in range(_XCH):
        sl = pl.ds(i * (nG // _XCH), nG // _XCH)
        pltpu.make_async_copy(x_hbm.at[sl], x_vmem.at[sl], in_sems.at[i]).wait()
    pltpu.make_async_copy(adj_hbm, adj_vmem, adj_sem).wait()

    A = adj_vmem[...]
    Ab = (A != 0).astype(jnp.float32)
    deg = jnp.sum(Ab, axis=0)
    deg_inv = jnp.where(deg > 0, 1.0 / jnp.maximum(deg, 1.0), 0.0)
    A_s = Ab * deg_inv[None, :]

    slabs = []
    for g in range(nG):
        slabs.append(jnp.pad(x_vmem[g], ((0, 0), (0, _LPAD - L))))
    Xn = jnp.concatenate(slabs, axis=1)

    def dotT(Lhs, H):
        return jax.lax.dot_general(
            Lhs, H, (((0,), (0,)), ((), ())),
            preferred_element_type=jnp.float32)

    def lin(H, W):
        cols = [
            jnp.dot(H[:, 128 * t:128 * (t + 1)], W,
                    preferred_element_type=jnp.float32)
            for t in range(_NTILE)
        ]
        return jnp.concatenate(cols, axis=1)

    U = 4.0 * Xn
    V = dotT(A, Xn)
    for i in range(_NLAYER):
        Ws = ws_ref[i]
        Wn = wn_ref[i]
        bias = b_ref[i]
        AU = dotT(A_s, U)
        AV = dotT(A_s, V)
        U = lin(U, Ws) + lin(AU, Wn) + bias[None, :]
        V = lin(V, Ws) + lin(AV, Wn) + bias[None, :]

    for c in range(nC):
        for b in range(nS):
            g = b * nC + c
            out_vmem[b, 2 * c] = U[:, _LPAD * g:_LPAD * g + L]
            out_vmem[b, 2 * c + 1] = V[:, _LPAD * g:_LPAD * g + L]

    nco = 2 * nC // (_OCH // nS)
    for i in range(_OCH):
        b = i // (_OCH // nS)
        j = i % (_OCH // nS)
        sl = pl.ds(j * nco, nco)
        pltpu.make_async_copy(out_vmem.at[b, sl], out_hbm.at[b, sl],
                              out_sems.at[i]).start()
    for i in range(_OCH):
        b = i // (_OCH // nS)
        j = i % (_OCH // nS)
        sl = pl.ds(j * nco, nco)
        pltpu.make_async_copy(out_vmem.at[b, sl], out_hbm.at[b, sl],
                              out_sems.at[i]).wait()


def kernel(x, adj, W_self, b_self, W_neigh):
    nS, nC, nN, L = x.shape
    nG = nC * nS

    def mk_tiles(W):
        Wp = jnp.pad(jnp.swapaxes(W, 1, 2),
                     ((0, 0), (0, _LPAD - L), (0, _LPAD - L)))
        z = jnp.zeros_like(Wp)
        rows = [jnp.concatenate([Wp if c == r else z for c in range(4)], axis=2)
                for r in range(4)]
        return jnp.concatenate(rows, axis=1)

    Wst = mk_tiles(W_self)
    Wnt = mk_tiles(W_neigh)
    bt = jnp.tile(jnp.pad(b_self, ((0, 0), (0, _LPAD - L))), (1, nG))

    xg = x.reshape(nS * nC, nN, L)

    out = pl.pallas_call(
        _gnn_body,
        in_specs=[
            pl.BlockSpec(memory_space=pl.ANY),
            pl.BlockSpec(memory_space=pl.ANY),
            pl.BlockSpec(memory_space=pltpu.VMEM),
            pl.BlockSpec(memory_space=pltpu.VMEM),
            pl.BlockSpec(memory_space=pltpu.VMEM),
        ],
        out_specs=pl.BlockSpec(memory_space=pl.ANY),
        out_shape=jax.ShapeDtypeStruct((nS, 2 * nC, nN, L), jnp.float32),
        scratch_shapes=[
            pltpu.VMEM((nS * nC, nN, L), jnp.float32),
            pltpu.VMEM((nN, nN), jnp.float32),
            pltpu.VMEM((nS, 2 * nC, nN, L), jnp.float32),
            pltpu.SemaphoreType.DMA((_XCH,)),
            pltpu.SemaphoreType.DMA,
            pltpu.SemaphoreType.DMA((_OCH,)),
        ],
    )(xg, adj, Wst, Wnt, bt)
    return out


# speedup vs baseline: 1.1579x; 1.1579x over previous
import jax
import jax.numpy as jnp
from jax.experimental import pallas as pl
from jax.experimental.pallas import tpu as pltpu

_NLAYER = 3
_L = 24
_LPAD = 32
_NTILE = 8
_GPT = 4


def _gnn_body(x_ref, adj_ref, ws_ref, wn_ref, b_ref, u_ref, v_ref, as_ref):
    i = pl.program_id(0)

    @pl.when(i == 0)
    def _():
        A0 = adj_ref[...]
        Ab = (A0 != 0).astype(jnp.float32)
        deg = jnp.sum(Ab, axis=0)
        deg_inv = jnp.where(deg > 0, 1.0 / jnp.maximum(deg, 1.0), 0.0)
        as_ref[...] = Ab * deg_inv[None, :]

    slabs = [jnp.pad(x_ref[j], ((0, 0), (0, _LPAD - _L))) for j in range(_GPT)]
    Xc = jnp.concatenate(slabs, axis=1)

    def dotT(Lhs, H):
        return jax.lax.dot_general(
            Lhs, H, (((0,), (0,)), ((), ())),
            preferred_element_type=jnp.float32)

    A = adj_ref[...]
    A_s = as_ref[...]
    U = 4.0 * Xc
    V = dotT(A, Xc)
    for l in range(_NLAYER):
        Ws = ws_ref[l]
        Wn = wn_ref[l]
        bias = b_ref[l]
        AU = dotT(A_s, U)
        AV = dotT(A_s, V)
        U = jnp.dot(U, Ws, preferred_element_type=jnp.float32) \
            + jnp.dot(AU, Wn, preferred_element_type=jnp.float32) + bias
        V = jnp.dot(V, Ws, preferred_element_type=jnp.float32) \
            + jnp.dot(AV, Wn, preferred_element_type=jnp.float32) + bias
    u_ref[...] = U
    v_ref[...] = V


def kernel(x, adj, W_self, b_self, W_neigh):
    nS, nC, nN, L = x.shape
    nG = nC * nS

    def mk_tiles(W):
        Wp = jnp.pad(jnp.swapaxes(W, 1, 2),
                     ((0, 0), (0, _LPAD - L), (0, _LPAD - L)))
        z = jnp.zeros_like(Wp)
        rows = [jnp.concatenate([Wp if c == r else z for c in range(4)], axis=2)
                for r in range(4)]
        return jnp.concatenate(rows, axis=1)

    Wst = mk_tiles(W_self)
    Wnt = mk_tiles(W_neigh)
    bp = jnp.pad(b_self, ((0, 0), (0, _LPAD - L)))
    bt = jnp.tile(bp, (1, _GPT)).reshape(_NLAYER, 1, _GPT * _LPAD)

    xg = x.reshape(nG, nN, L)

    U, V = pl.pallas_call(
        _gnn_body,
        grid=(_NTILE,),
        in_specs=[
            pl.BlockSpec((_GPT, nN, L), lambda i: (i, 0, 0)),
            pl.BlockSpec((nN, nN), lambda i: (0, 0)),
            pl.BlockSpec((_NLAYER, 128, 128), lambda i: (0, 0, 0)),
            pl.BlockSpec((_NLAYER, 128, 128), lambda i: (0, 0, 0)),
            pl.BlockSpec((_NLAYER, 1, 128), lambda i: (0, 0, 0)),
        ],
        out_specs=[
            pl.BlockSpec((nN, 128), lambda i: (0, i)),
            pl.BlockSpec((nN, 128), lambda i: (0, i)),
        ],
        out_shape=[
            jax.ShapeDtypeStruct((nN, nG * _LPAD), jnp.float32),
            jax.ShapeDtypeStruct((nN, nG * _LPAD), jnp.float32),
        ],
        scratch_shapes=[pltpu.VMEM((nN, nN), jnp.float32)],
    )(xg, adj, Wst, Wnt, bt)

    Ur = U.reshape(nN, nS, nC, _LPAD)[..., :L].transpose(1, 2, 0, 3)
    Vr = V.reshape(nN, nS, nC, _LPAD)[..., :L].transpose(1, 2, 0, 3)
    out = jnp.stack([Ur, Vr], axis=2).reshape(nS, 2 * nC, nN, L)
    return out
